# baseline (speedup 1.0000x reference)
import functools

import jax
import jax.numpy as jnp
import numpy as np
from jax.experimental import pallas as pl
from jax.experimental.pallas import tpu as pltpu

EMB = 64
HEADS = 2
EPS_LN = 1e-5


def _mlp_body(x_ref, w1_ref, b1_ref, w2_ref, b2_ref, o_ref):
    h = jnp.dot(x_ref[...], w1_ref[...], preferred_element_type=jnp.float32)
    h = h + b1_ref[...]
    h = jnp.where(h > 0, h, 0.01 * h)
    o_ref[...] = jnp.dot(h, w2_ref[...], preferred_element_type=jnp.float32) + b2_ref[...]


def _mlp(x, W1, b1, W2, b2):
    n, d = x.shape
    d4 = W1.shape[1]
    BLK = 1024
    npad = (n + BLK - 1) // BLK * BLK
    xp = jnp.pad(x, ((0, npad - n), (0, 0)))
    out = pl.pallas_call(
        _mlp_body,
        grid=(npad // BLK,),
        in_specs=[
            pl.BlockSpec((BLK, d), lambda i: (i, 0)),
            pl.BlockSpec((d, d4), lambda i: (0, 0)),
            pl.BlockSpec((1, d4), lambda i: (0, 0)),
            pl.BlockSpec((d4, d), lambda i: (0, 0)),
            pl.BlockSpec((1, d), lambda i: (0, 0)),
        ],
        out_specs=pl.BlockSpec((BLK, d), lambda i: (i, 0)),
        out_shape=jax.ShapeDtypeStruct((npad, d), jnp.float32),
    )(xp, W1, b1.reshape(1, -1), W2, b2.reshape(1, -1))
    return out[:n]


def _graph_layernorm(x, batch, num_graphs):
    d = x.shape[-1]
    cnt = jax.ops.segment_sum(jnp.ones((x.shape[0],), x.dtype), batch, num_graphs) * d
    cnt = jnp.maximum(cnt, 1.0)
    mean = jax.ops.segment_sum(x.sum(-1), batch, num_graphs) / cnt
    xc = x - mean[batch][:, None]
    var = jax.ops.segment_sum((xc * xc).sum(-1), batch, num_graphs) / cnt
    return xc / jnp.sqrt(var + EPS_LN)[batch][:, None]


def _gnn_forward(params, x, edge_index, edge_attr, batch, num_graphs):
    src, dst = edge_index[0], edge_index[1]
    n = x.shape[0]
    for p in params:
        x_norm = _graph_layernorm(x, batch, num_graphs)
        msg = jax.nn.relu(x_norm[src] + edge_attr) + 1e-7
        agg = jax.ops.segment_sum(msg, dst, n)
        gen_out = (agg + x_norm) @ p['gen_W'] + p['gen_b']
        xt = jnp.concatenate([x_norm, gen_out], axis=1)
        q = (xt @ p['Wq'] + p['bq']).reshape(n, HEADS, EMB)
        k = (xt @ p['Wk'] + p['bk']).reshape(n, HEADS, EMB)
        v = (xt @ p['Wv'] + p['bv']).reshape(n, HEADS, EMB)
        eproj = (edge_attr @ p['We']).reshape(-1, HEADS, EMB)
        k_j = k[src] + eproj
        alpha = (q[dst] * k_j).sum(-1) / np.sqrt(EMB)
        amax = jax.ops.segment_max(alpha, dst, n)
        amax = jnp.where(jnp.isfinite(amax), amax, 0.0)
        aexp = jnp.exp(alpha - amax[dst])
        denom = jax.ops.segment_sum(aexp, dst, n)
        alpha_n = aexp / jnp.maximum(denom[dst], 1e-16)
        tmsg = (v[src] + eproj) * alpha_n[:, :, None]
        tout = jax.ops.segment_sum(tmsg, dst, n).reshape(n, HEADS * EMB)
        tout = tout + (xt @ p['Wskip'] + p['bskip'])
        l_h = tout @ p['Wlin'] + p['blin']
        h = _graph_layernorm(l_h, batch, num_graphs)
        h = _mlp(h, p['W1'], p['b1'], p['W2'], p['b2'])
        x = x + h
    return x


def kernel(x, edge_attr, cond, params, edge_index, batch, non_edge_index):
    N = x.shape[0]
    G = cond.shape[0]
    x_aug = jnp.concatenate([x, cond], axis=0)
    u = jnp.arange(N, dtype=batch.dtype)
    v = batch + N
    aug_ei = jnp.concatenate([edge_index, jnp.stack([u, v]), jnp.stack([v, u])], axis=1)
    e_p = jnp.zeros((2 * N, edge_attr.shape[1]), jnp.float32).at[:, 0].set(1.0)
    aug_e = jnp.concatenate([edge_attr, e_p], axis=0)
    Nt = N + G
    dst = aug_ei[1]
    cnt = jax.ops.segment_sum(jnp.ones((dst.shape[0],), jnp.float32), dst, Nt)
    loop_attr = jax.ops.segment_sum(aug_e, dst, Nt) / jnp.maximum(cnt, 1.0)[:, None]
    loops = jnp.arange(Nt, dtype=batch.dtype)
    aug_ei = jnp.concatenate([aug_ei, jnp.stack([loops, loops])], axis=1)
    aug_e = jnp.concatenate([aug_e, loop_attr], axis=0)
    aug_batch = jnp.concatenate([batch, jnp.arange(G, dtype=batch.dtype)], axis=0)
    h_aug = _gnn_forward(params, x_aug, aug_ei, aug_e, aug_batch, G)
    n_emb = h_aug[:N]
    v_emb = h_aug[N:]
    cntn = jnp.maximum(jax.ops.segment_sum(jnp.ones((N,), jnp.float32), batch, G), 1.0)
    glob = jax.ops.segment_sum(n_emb, batch, G) / cntn[:, None] + v_emb
    ne_row, ne_col = non_edge_index[0], non_edge_index[1]
    ne_emb = n_emb[ne_row] + n_emb[ne_col]
    return n_emb, glob, ne_emb


# speedup vs baseline: 6.6336x; 6.6336x over previous
import functools

import jax
import jax.numpy as jnp
import numpy as np
from jax import lax
from jax.experimental import pallas as pl
from jax.experimental.pallas import tpu as pltpu
from jax.experimental.pallas import tpu_sc as plsc

EMB = 64
HEADS = 2
EPS_LN = 1e-5
AW = 8
NTP = 51200
G_MAX = 512

_SCI = plsc.get_sparse_core_info()
_NC, _NS = _SCI.num_cores, _SCI.num_subcores
_NW = _NC * _NS
_SC_MESH = dict(core_axis_name="c", subcore_axis_name="s")
_SC_PARAMS = pltpu.CompilerParams(use_tc_tiling_on_sc=False)


def _wid():
    return lax.axis_index("s") * _NC + lax.axis_index("c")


def _sc_gather(table, idx2):
    M, D = table.shape
    NCH, IB = idx2.shape
    nchunk = NCH // _NW

    @functools.partial(
        pl.kernel,
        out_type=jax.ShapeDtypeStruct((NCH * IB, D), jnp.float32),
        mesh=plsc.VectorSubcoreMesh(**_SC_MESH),
        compiler_params=_SC_PARAMS,
        scratch_types=[
            pltpu.VMEM((nchunk, IB), jnp.int32),
            pltpu.VMEM((IB, D), jnp.float32),
            pltpu.VMEM((IB, D), jnp.float32),
            pltpu.SemaphoreType.DMA,
            pltpu.SemaphoreType.DMA,
        ],
    )
    def k(tab_hbm, idx_hbm, out_hbm, idx_v, ra_v, rb_v, sem_a, sem_b):
        w = _wid()
        c0 = w * nchunk
        pltpu.sync_copy(idx_hbm.at[pl.ds(c0, nchunk)], idx_v)
        cp0 = pltpu.async_copy(tab_hbm.at[idx_v.at[0]], ra_v, sem_a)

        def body(j, carry):
            nxt = jnp.minimum(j + 1, nchunk - 1)

            @pl.when(j % 2 == 0)
            def _():
                pltpu.async_copy(tab_hbm.at[idx_v.at[nxt]], rb_v, sem_b)
                pltpu.make_async_copy(tab_hbm.at[idx_v.at[0]], ra_v, sem_a).wait()
                pltpu.sync_copy(ra_v, out_hbm.at[pl.ds((c0 + j) * IB, IB)])

            @pl.when(j % 2 == 1)
            def _():
                pltpu.async_copy(tab_hbm.at[idx_v.at[nxt]], ra_v, sem_a)
                pltpu.make_async_copy(tab_hbm.at[idx_v.at[0]], rb_v, sem_b).wait()
                pltpu.sync_copy(rb_v, out_hbm.at[pl.ds((c0 + j) * IB, IB)])

            return carry

        lax.fori_loop(0, nchunk - 1, body, 0)
        j = nchunk - 1

        @pl.when(j % 2 == 0)
        def _():
            pltpu.make_async_copy(tab_hbm.at[idx_v.at[0]], ra_v, sem_a).wait()
            pltpu.sync_copy(ra_v, out_hbm.at[pl.ds((c0 + j) * IB, IB)])

        @pl.when(j % 2 == 1)
        def _():
            pltpu.make_async_copy(tab_hbm.at[idx_v.at[0]], rb_v, sem_b).wait()
            pltpu.sync_copy(rb_v, out_hbm.at[pl.ds((c0 + j) * IB, IB)])

        _ = cp0

    return k(table, idx2)


def _sc_gather2_add(table, idx2a, idx2b):
    M, D = table.shape
    NCH, IB = idx2a.shape
    nchunk = NCH // _NW

    @functools.partial(
        pl.kernel,
        out_type=jax.ShapeDtypeStruct((NCH * IB, D), jnp.float32),
        mesh=plsc.VectorSubcoreMesh(**_SC_MESH),
        compiler_params=_SC_PARAMS,
        scratch_types=[
            pltpu.VMEM((nchunk, IB), jnp.int32),
            pltpu.VMEM((nchunk, IB), jnp.int32),
            pltpu.VMEM((IB, D), jnp.float32),
            pltpu.VMEM((IB, D), jnp.float32),
            pltpu.SemaphoreType.DMA,
            pltpu.SemaphoreType.DMA,
        ],
    )
    def k(tab_hbm, ia_hbm, ib_hbm, out_hbm, ia_v, ib_v, ra_v, rb_v, sem_a, sem_b):
        w = _wid()
        c0 = w * nchunk
        pltpu.sync_copy(ia_hbm.at[pl.ds(c0, nchunk)], ia_v)
        pltpu.sync_copy(ib_hbm.at[pl.ds(c0, nchunk)], ib_v)

        def body(j, carry):
            cp_a = pltpu.async_copy(tab_hbm.at[ia_v.at[j]], ra_v, sem_a)
            cp_b = pltpu.async_copy(tab_hbm.at[ib_v.at[j]], rb_v, sem_b)
            cp_a.wait()
            cp_b.wait()

            def add_row(i, c):
                for t in range(D // 16):
                    s = pl.ds(t * 16, 16)
                    ra_v[i, s] = ra_v[i, s] + rb_v[i, s]
                return c

            lax.fori_loop(0, IB, add_row, 0)
            pltpu.sync_copy(ra_v, out_hbm.at[pl.ds((c0 + j) * IB, IB)])
            return carry

        lax.fori_loop(0, nchunk, body, 0)

    return k(table, idx2a, idx2b)


def _sc_segsum(dataf, edst, bounds, D, NR, RB, CC):
    NRP = bounds.shape[0]
    NRW = NR // _NW
    ACC = (RB + 1) * D

    @functools.partial(
        pl.kernel,
        out_type=jax.ShapeDtypeStruct((NTP * D,), jnp.float32),
        mesh=plsc.VectorSubcoreMesh(**_SC_MESH),
        compiler_params=_SC_PARAMS,
        scratch_types=[
            pltpu.VMEM((NRP + 16,), jnp.int32),
            pltpu.VMEM((CC + 16,), jnp.int32),
            pltpu.VMEM((CC * D,), jnp.float32),
            pltpu.VMEM((ACC,), jnp.float32),
        ],
    )
    def k(dat_hbm, dst_hbm, bnd_hbm, out_hbm, bnd_v, dst_v, dat_v, acc_v):
        w = _wid()
        pltpu.sync_copy(bnd_hbm, bnd_v.at[pl.ds(0, NRP)])
        zero = jnp.zeros((16,), jnp.float32)
        for rr in range(NRW):
            r = w * NRW + rr
            d0 = r * RB

            def zbody(i, c):
                acc_v[pl.ds(i * 16, 16)] = zero
                return c

            lax.fori_loop(0, ACC // 16, zbody, 0)
            e0 = bnd_v[pl.ds(r, 16)][0]
            e1 = bnd_v[pl.ds(r + 1, 16)][0]
            e0a = (e0 // 8) * 8
            nch = (e1 - e0a + CC - 1) // CC

            def chunk(c, carry):
                eo = e0a + c * CC
                pltpu.sync_copy(dst_hbm.at[pl.ds(eo, CC)], dst_v.at[pl.ds(0, CC)])
                pltpu.sync_copy(dat_hbm.at[pl.ds(eo * D, CC * D)], dat_v)

                def edge(i, cc):
                    dv = dst_v[pl.ds(i, 16)][0]
                    e = eo + i
                    valid = (e >= e0) & (e < e1)
                    dl = jnp.where(valid, dv - d0, RB)
                    for t in range(D // 16):
                        vvec = dat_v[pl.ds(i * D + t * 16, 16)]
                        plsc.addupdate(acc_v.at[pl.ds(dl * D + t * 16, 16)], vvec)
                    return cc

                lax.fori_loop(0, CC, edge, 0)
                return carry

            lax.fori_loop(0, nch, chunk, 0)
            pltpu.sync_copy(acc_v.at[pl.ds(0, RB * D)],
                            out_hbm.at[pl.ds(d0 * D, RB * D)])

    return k(dataf, edst, bounds)


def _sc_segstat(alphaf, edst, bounds, amaxf, NR, RB, CC, mode):
    NRP = bounds.shape[0]
    NRW = NR // _NW
    ACC = (RB + 2) * AW + 16

    @functools.partial(
        pl.kernel,
        out_type=jax.ShapeDtypeStruct((NTP * AW,), jnp.float32),
        mesh=plsc.VectorSubcoreMesh(**_SC_MESH),
        compiler_params=_SC_PARAMS,
        scratch_types=[
            pltpu.VMEM((NRP + 16,), jnp.int32),
            pltpu.VMEM((CC + 16,), jnp.int32),
            pltpu.VMEM((CC * AW + 16,), jnp.float32),
            pltpu.VMEM((ACC,), jnp.float32),
            pltpu.VMEM((RB * AW + 16,), jnp.float32),
        ],
    )
    def k(a_hbm, dst_hbm, bnd_hbm, amx_hbm, out_hbm,
          bnd_v, dst_v, dat_v, acc_v, amx_v):
        w = _wid()
        pltpu.sync_copy(bnd_hbm, bnd_v.at[pl.ds(0, NRP)])
        init = jnp.full((16,), -1e30 if mode == 'max' else 0.0, jnp.float32)
        maskf = jnp.where(lax.broadcasted_iota(jnp.int32, (16,), 0) < HEADS, 1.0, 0.0)
        for rr in range(NRW):
            r = w * NRW + rr
            d0 = r * RB

            def zbody(i, c):
                acc_v[pl.ds(i * 16, 16)] = init
                return c

            lax.fori_loop(0, ACC // 16, zbody, 0)
            if mode == 'expsum':
                pltpu.sync_copy(amx_hbm.at[pl.ds(d0 * AW, RB * AW)],
                                amx_v.at[pl.ds(0, RB * AW)])
            e0 = bnd_v[pl.ds(r, 16)][0]
            e1 = bnd_v[pl.ds(r + 1, 16)][0]
            e0a = (e0 // 8) * 8
            nch = (e1 - e0a + CC - 1) // CC

            def chunk(c, carry):
                eo = e0a + c * CC
                pltpu.sync_copy(dst_hbm.at[pl.ds(eo, CC)], dst_v.at[pl.ds(0, CC)])
                pltpu.sync_copy(a_hbm.at[pl.ds(eo * AW, CC * AW)],
                                dat_v.at[pl.ds(0, CC * AW)])

                def edge(i, cc):
                    dv = dst_v[pl.ds(i, 16)][0]
                    e = eo + i
                    valid = (e >= e0) & (e < e1)
                    dl = jnp.where(valid, dv - d0, RB)
                    av = dat_v[pl.ds(i * AW, 16)]
                    va = acc_v[pl.ds(dl * AW, 16)]
                    validf = jnp.where(valid, 1.0, 0.0)
                    m = maskf * validf
                    if mode == 'expsum':
                        mxv = amx_v[pl.ds(dl * AW, 16)]
                        acc_v[pl.ds(dl * AW, 16)] = va + jnp.exp(av - mxv) * m
                    else:
                        sel = m > 0.5
                        acc_v[pl.ds(dl * AW, 16)] = jnp.where(sel, jnp.maximum(va, av), va)
                    return cc

                lax.fori_loop(0, CC, edge, 0)
                return carry

            lax.fori_loop(0, nch, chunk, 0)
            pltpu.sync_copy(acc_v.at[pl.ds(0, RB * AW)],
                            out_hbm.at[pl.ds(d0 * AW, RB * AW)])

    return k(alphaf, edst, bounds, amaxf)


_BN = 512
_BE = 1024


def _onehot(batch_ref):
    b = batch_ref[0, 0, :]
    cols = lax.broadcasted_iota(jnp.int32, (_BN, G_MAX), 1)
    return jnp.where(b[:, None] == cols, 1.0, 0.0).astype(jnp.float32)


def _lnstats_body(x_ref, b_ref, o_ref):
    i = pl.program_id(0)
    oh = _onehot(b_ref)
    x = x_ref[...]
    s1 = jnp.sum(x, axis=1, keepdims=True)
    s2 = jnp.sum(x * x, axis=1, keepdims=True)
    S = jnp.concatenate([s1, s2, jnp.zeros((_BN, 6), jnp.float32)], axis=1)
    P = lax.dot_general(oh, S, (((0,), (0,)), ((), ())),
                        preferred_element_type=jnp.float32)

    @pl.when(i == 0)
    def _():
        o_ref[...] = jnp.zeros_like(o_ref)

    o_ref[...] += P


def _lnstats(x, batch3):
    return pl.pallas_call(
        _lnstats_body, grid=(x.shape[0] // _BN,),
        in_specs=[pl.BlockSpec((_BN, x.shape[1]), lambda i: (i, 0)),
                  pl.BlockSpec((1, 1, _BN), lambda i: (i, 0, 0))],
        out_specs=pl.BlockSpec((G_MAX, 8), lambda i: (0, 0)),
        out_shape=jax.ShapeDtypeStruct((G_MAX, 8), jnp.float32))(x, batch3)


def _pool_body(x_ref, b_ref, o_ref):
    i = pl.program_id(0)
    oh = _onehot(b_ref)
    P = lax.dot_general(oh, x_ref[...], (((0,), (0,)), ((), ())),
                        preferred_element_type=jnp.float32)

    @pl.when(i == 0)
    def _():
        o_ref[...] = jnp.zeros_like(o_ref)

    o_ref[...] += P


def _pool(x, batch3):
    return pl.pallas_call(
        _pool_body, grid=(x.shape[0] // _BN,),
        in_specs=[pl.BlockSpec((_BN, 64), lambda i: (i, 0)),
                  pl.BlockSpec((1, 1, _BN), lambda i: (i, 0, 0))],
        out_specs=pl.BlockSpec((G_MAX, 64), lambda i: (0, 0)),
        out_shape=jax.ShapeDtypeStruct((G_MAX, 64), jnp.float32))(x, batch3)


def _lnapply_body(x_ref, b_ref, st_ref, o_ref):
    oh = _onehot(b_ref)
    m = jnp.dot(oh, st_ref[...], preferred_element_type=jnp.float32)
    o_ref[...] = (x_ref[...] - m[:, 0:1]) * m[:, 1:2]


def _lnapply(x, batch3, stats8):
    return pl.pallas_call(
        _lnapply_body, grid=(x.shape[0] // _BN,),
        in_specs=[pl.BlockSpec((_BN, 64), lambda i: (i, 0)),
                  pl.BlockSpec((1, 1, _BN), lambda i: (i, 0, 0)),
                  pl.BlockSpec((G_MAX, 8), lambda i: (0, 0))],
        out_specs=pl.BlockSpec((_BN, 64), lambda i: (i, 0)),
        out_shape=jax.ShapeDtypeStruct((x.shape[0], 64), jnp.float32))(x, batch3, stats8)


def _dense1_body(xn_ref, aggs_ref, las_ref, rc_ref,
                 genW_ref, genb_ref, Wq_ref, bq_ref, Wk_ref, bk_ref,
                 Wv_ref, bv_ref, Wsk_ref, bsk_ref, We_ref,
                 q_ref, kv_ref, sk_ref, ep_ref, as_ref):
    xn = xn_ref[...]
    la = las_ref[...] * rc_ref[..., 0:1]
    agg = aggs_ref[...] + jax.nn.relu(xn + la) + 1e-7
    gen = jnp.dot(agg + xn, genW_ref[...],
                  preferred_element_type=jnp.float32) + genb_ref[...]

    def two(W_ref, b_ref):
        return (jnp.dot(xn, W_ref[0:64, :], preferred_element_type=jnp.float32)
                + jnp.dot(gen, W_ref[64:128, :], preferred_element_type=jnp.float32)
                + b_ref[...])

    q = two(Wq_ref, bq_ref)
    k = two(Wk_ref, bk_ref)
    v = two(Wv_ref, bv_ref)
    sk = two(Wsk_ref, bsk_ref)
    ep = jnp.dot(la, We_ref[...], preferred_element_type=jnp.float32)
    q_ref[...] = q
    kv_ref[...] = jnp.concatenate([k, v], axis=1)
    sk_ref[...] = sk
    ep_ref[...] = ep
    kk = k + ep
    a0 = jnp.sum(q[:, 0:64] * kk[:, 0:64], axis=1, keepdims=True) * 0.125
    a1 = jnp.sum(q[:, 64:128] * kk[:, 64:128], axis=1, keepdims=True) * 0.125
    z = jnp.zeros_like(a0)
    as_ref[...] = jnp.concatenate([a0, a1, z, z, z, z, z, z], axis=1)


def _dense1(xn, aggs, las, rcnt8, p):
    n = xn.shape[0]

    def wspec(shp):
        return pl.BlockSpec(shp, lambda i: (0, 0))

    return pl.pallas_call(
        _dense1_body, grid=(n // _BN,),
        in_specs=[pl.BlockSpec((_BN, 64), lambda i: (i, 0)),
                  pl.BlockSpec((_BN, 64), lambda i: (i, 0)),
                  pl.BlockSpec((_BN, 64), lambda i: (i, 0)),
                  pl.BlockSpec((_BN, 8), lambda i: (i, 0)),
                  wspec((64, 64)), wspec((1, 64)),
                  wspec((128, 128)), wspec((1, 128)),
                  wspec((128, 128)), wspec((1, 128)),
                  wspec((128, 128)), wspec((1, 128)),
                  wspec((128, 128)), wspec((1, 128)),
                  wspec((64, 128))],
        out_specs=[pl.BlockSpec((_BN, 128), lambda i: (i, 0)),
                   pl.BlockSpec((_BN, 256), lambda i: (i, 0)),
                   pl.BlockSpec((_BN, 128), lambda i: (i, 0)),
                   pl.BlockSpec((_BN, 128), lambda i: (i, 0)),
                   pl.BlockSpec((_BN, 8), lambda i: (i, 0))],
        out_shape=[jax.ShapeDtypeStruct((n, 128), jnp.float32),
                   jax.ShapeDtypeStruct((n, 256), jnp.float32),
                   jax.ShapeDtypeStruct((n, 128), jnp.float32),
                   jax.ShapeDtypeStruct((n, 128), jnp.float32),
                   jax.ShapeDtypeStruct((n, 8), jnp.float32)])(
            xn, aggs, las, rcnt8,
            p['gen_W'], p['gen_b'].reshape(1, -1),
            p['Wq'], p['bq'].reshape(1, -1), p['Wk'], p['bk'].reshape(1, -1),
            p['Wv'], p['bv'].reshape(1, -1), p['Wskip'], p['bskip'].reshape(1, -1),
            p['We'])


def _eproj_body(e_ref, We_ref, o_ref):
    o_ref[...] = jnp.dot(e_ref[...], We_ref[...], preferred_element_type=jnp.float32)


def _eproj(eattr, We):
    n = eattr.shape[0]
    return pl.pallas_call(
        _eproj_body, grid=(n // _BE,),
        in_specs=[pl.BlockSpec((_BE, 64), lambda i: (i, 0)),
                  pl.BlockSpec((64, 128), lambda i: (0, 0))],
        out_specs=pl.BlockSpec((_BE, 128), lambda i: (i, 0)),
        out_shape=jax.ShapeDtypeStruct((n, 128), jnp.float32))(eattr, We)


def _fixattr_body(g_ref, p_ref, o_ref):
    pidx = p_ref[0, 0, :]
    isreal = jnp.where(pidx < _E_REAL, 1.0, 0.0)[:, None]
    ep_row = jnp.where(lax.broadcasted_iota(jnp.int32, (_BE, 64), 1) == 0, 1.0, 0.0)
    o_ref[...] = g_ref[...] * isreal + ep_row * (1.0 - isreal)


def _fixattr(gathered, perm3):
    n = gathered.shape[0]
    return pl.pallas_call(
        _fixattr_body, grid=(n // _BE,),
        in_specs=[pl.BlockSpec((_BE, 64), lambda i: (i, 0)),
                  pl.BlockSpec((1, 1, _BE), lambda i: (i, 0, 0))],
        out_specs=pl.BlockSpec((_BE, 64), lambda i: (i, 0)),
        out_shape=jax.ShapeDtypeStruct((n, 64), jnp.float32))(gathered, perm3)


def _msg_body(xs_ref, e_ref, o_ref):
    o_ref[...] = jax.nn.relu(xs_ref[...] + e_ref[...]) + 1e-7


def _msg(xs, eattr):
    n = xs.shape[0]
    return pl.pallas_call(
        _msg_body, grid=(n // _BE,),
        in_specs=[pl.BlockSpec((_BE, 64), lambda i: (i, 0))] * 2,
        out_specs=pl.BlockSpec((_BE, 64), lambda i: (i, 0)),
        out_shape=jax.ShapeDtypeStruct((n, 64), jnp.float32))(xs, eattr)


def _alpha_body(ks_ref, qd_ref, ep_ref, o_ref):
    kk = ks_ref[...] + ep_ref[...]
    q = qd_ref[...]
    a0 = jnp.sum(q[:, 0:64] * kk[:, 0:64], axis=1, keepdims=True) * 0.125
    a1 = jnp.sum(q[:, 64:128] * kk[:, 64:128], axis=1, keepdims=True) * 0.125
    z = jnp.zeros_like(a0)
    o_ref[...] = jnp.concatenate([a0, a1, z, z, z, z, z, z], axis=1)


def _alpha(kvs, qd, ep):
    n = kvs.shape[0]
    return pl.pallas_call(
        _alpha_body, grid=(n // _BE,),
        in_specs=[pl.BlockSpec((_BE, 128), lambda i: (i, 0))] * 3,
        out_specs=pl.BlockSpec((_BE, 8), lambda i: (i, 0)),
        out_shape=jax.ShapeDtypeStruct((n, 8), jnp.float32))(kvs, qd, ep)


def _mergeA_body(mx_ref, as_ref, o_ref):
    o_ref[...] = jnp.maximum(mx_ref[...], as_ref[...])


def _mergeA(mx8, aself8):
    n = mx8.shape[0]
    return pl.pallas_call(
        _mergeA_body, grid=(n // _BN,),
        in_specs=[pl.BlockSpec((_BN, 8), lambda i: (i, 0))] * 2,
        out_specs=pl.BlockSpec((_BN, 8), lambda i: (i, 0)),
        out_shape=jax.ShapeDtypeStruct((n, 8), jnp.float32))(mx8, aself8)


def _mergeB_body(dn_ref, as_ref, am_ref, sm_ref, an_ref):
    am = am_ref[...]
    es = jnp.exp(as_ref[...] - am)
    den = dn_ref[...] + es
    rden = 1.0 / jnp.maximum(den, 1e-16)
    sm = jnp.concatenate([am[:, 0:2], rden[:, 0:2], am[:, 4:8]], axis=1)
    an_ref[...] = es * rden
    sm_ref[...] = sm


def _mergeB(dens8, aself8, amax8):
    n = dens8.shape[0]
    return pl.pallas_call(
        _mergeB_body, grid=(n // _BN,),
        in_specs=[pl.BlockSpec((_BN, 8), lambda i: (i, 0))] * 3,
        out_specs=[pl.BlockSpec((_BN, 8), lambda i: (i, 0))] * 2,
        out_shape=[jax.ShapeDtypeStruct((n, 8), jnp.float32)] * 2)(dens8, aself8, amax8)


def _tmsg_body(vs_ref, ep_ref, a_ref, sm_ref, o_ref):
    vv = vs_ref[...] + ep_ref[...]
    a = a_ref[...]
    sm = sm_ref[...]
    w0 = jnp.exp(a[:, 0:1] - sm[:, 0:1]) * sm[:, 2:3]
    w1 = jnp.exp(a[:, 1:2] - sm[:, 1:2]) * sm[:, 3:4]
    o_ref[...] = jnp.concatenate([vv[:, 0:64] * w0, vv[:, 64:128] * w1], axis=1)


def _tmsg(kvs, ep, alpha8, smE):
    n = kvs.shape[0]
    return pl.pallas_call(
        _tmsg_body, grid=(n // _BE,),
        in_specs=[pl.BlockSpec((_BE, 128), lambda i: (i, 1)),
                  pl.BlockSpec((_BE, 128), lambda i: (i, 0)),
                  pl.BlockSpec((_BE, 8), lambda i: (i, 0)),
                  pl.BlockSpec((_BE, 8), lambda i: (i, 0))],
        out_specs=pl.BlockSpec((_BE, 128), lambda i: (i, 0)),
        out_shape=jax.ShapeDtypeStruct((n, 128), jnp.float32))(kvs, ep, alpha8, smE)


def _tout_body(ts_ref, v_ref, ep_ref, sk_ref, an_ref, Wl_ref, bl_ref, o_ref):
    an = an_ref[...]
    vv = v_ref[...] + ep_ref[...]
    selfmsg = jnp.concatenate([vv[:, 0:64] * an[:, 0:1],
                               vv[:, 64:128] * an[:, 1:2]], axis=1)
    tout = ts_ref[...] + selfmsg + sk_ref[...]
    o_ref[...] = jnp.dot(tout, Wl_ref[...],
                         preferred_element_type=jnp.float32) + bl_ref[...]


def _tout(touts, kv, epself, sk, anself8, Wlin, blin):
    n = touts.shape[0]
    return pl.pallas_call(
        _tout_body, grid=(n // _BN,),
        in_specs=[pl.BlockSpec((_BN, 128), lambda i: (i, 0)),
                  pl.BlockSpec((_BN, 128), lambda i: (i, 1)),
                  pl.BlockSpec((_BN, 128), lambda i: (i, 0)),
                  pl.BlockSpec((_BN, 128), lambda i: (i, 0)),
                  pl.BlockSpec((_BN, 8), lambda i: (i, 0)),
                  pl.BlockSpec((128, 64), lambda i: (0, 0)),
                  pl.BlockSpec((1, 64), lambda i: (0, 0))],
        out_specs=pl.BlockSpec((_BN, 64), lambda i: (i, 0)),
        out_shape=jax.ShapeDtypeStruct((n, 64), jnp.float32))(
            touts, kv, epself, sk, anself8, Wlin, blin.reshape(1, -1))


def _mlpres_body(l_ref, b_ref, st_ref, x_ref, W1_ref, b1_ref, W2_ref, b2_ref, o_ref):
    oh = _onehot(b_ref)
    m = jnp.dot(oh, st_ref[...], preferred_element_type=jnp.float32)
    xn = (l_ref[...] - m[:, 0:1]) * m[:, 1:2]
    h = jnp.dot(xn, W1_ref[...], preferred_element_type=jnp.float32) + b1_ref[...]
    h = jnp.where(h > 0, h, 0.01 * h)
    o_ref[...] = (x_ref[...] + jnp.dot(h, W2_ref[...],
                                       preferred_element_type=jnp.float32) + b2_ref[...])


def _mlpres(l, batch3, stats8, x, p):
    n = l.shape[0]
    return pl.pallas_call(
        _mlpres_body, grid=(n // _BN,),
        in_specs=[pl.BlockSpec((_BN, 64), lambda i: (i, 0)),
                  pl.BlockSpec((1, 1, _BN), lambda i: (i, 0, 0)),
                  pl.BlockSpec((G_MAX, 8), lambda i: (0, 0)),
                  pl.BlockSpec((_BN, 64), lambda i: (i, 0)),
                  pl.BlockSpec((64, 256), lambda i: (0, 0)),
                  pl.BlockSpec((1, 256), lambda i: (0, 0)),
                  pl.BlockSpec((256, 64), lambda i: (0, 0)),
                  pl.BlockSpec((1, 64), lambda i: (0, 0))],
        out_specs=pl.BlockSpec((_BN, 64), lambda i: (i, 0)),
        out_shape=jax.ShapeDtypeStruct((n, 64), jnp.float32))(
            l, batch3, stats8, x, p['W1'], p['b1'].reshape(1, -1),
            p['W2'], p['b2'].reshape(1, -1))


_E_REAL = 800000
_ESP = 917504
_NEP = 819200


def _stats8(st, cnt_ln):
    mean = st[:, 0] / cnt_ln
    var = st[:, 1] / cnt_ln - mean * mean
    rstd = 1.0 / jnp.sqrt(var + EPS_LN)
    return jnp.concatenate(
        [mean[:, None], rstd[:, None], jnp.zeros((G_MAX, 6), jnp.float32)], axis=1)


def kernel(x, edge_attr, cond, params, edge_index, batch, non_edge_index):
    N, D = x.shape
    G = cond.shape[0]
    E = edge_index.shape[1]
    Nt = N + G
    Es = E + 2 * N
    idt = jnp.int32
    f32 = jnp.float32

    u = jnp.arange(N, dtype=idt)
    vvirt = (batch + N).astype(idt)
    asrc = jnp.concatenate([edge_index[0].astype(idt), u, vvirt])
    adst = jnp.concatenate([edge_index[1].astype(idt), vvirt, u])
    perm = jnp.argsort(adst).astype(idt)
    edst_s = adst[perm]
    esrc_s = asrc[perm]
    padn = _ESP - Es
    edst_sort = jnp.concatenate([edst_s, jnp.full((padn,), 2**30, idt)])
    edst_gidx = jnp.concatenate([edst_s, jnp.zeros((padn,), idt)])
    esrc_p = jnp.concatenate([esrc_s, jnp.zeros((padn,), idt)])
    perm_p = jnp.concatenate([perm, jnp.zeros((padn,), idt)])

    RB64, RB128 = NTP // 64, NTP // 128
    bnd64 = jnp.searchsorted(edst_sort, jnp.arange(65, dtype=idt) * RB64).astype(idt)
    bnd64 = jnp.concatenate([bnd64, jnp.full((7,), Es, idt)])
    bnd128 = jnp.searchsorted(edst_sort, jnp.arange(129, dtype=idt) * RB128).astype(idt)
    bnd128 = jnp.concatenate([bnd128, jnp.full((7,), Es, idt)])

    cum = jnp.searchsorted(edst_sort, jnp.arange(NTP + 1, dtype=idt)).astype(f32)
    cnt = cum[1:] - cum[:-1]
    rcnt = 1.0 / jnp.maximum(cnt, 1.0)
    rcnt8 = jnp.concatenate([rcnt[:, None], jnp.zeros((NTP, 7), f32)], axis=1)

    aug_batch = jnp.concatenate([batch.astype(idt), jnp.arange(G, dtype=idt)])
    batch_pad = jnp.concatenate([aug_batch, jnp.full((NTP - Nt,), G_MAX, idt)])
    batch3 = batch_pad.reshape(NTP // _BN, 1, _BN)
    NP2 = (N // _BN) * _BN + (_BN if N % _BN else 0)
    batchN = jnp.concatenate([batch.astype(idt), jnp.full((NP2 - N,), G_MAX, idt)])
    batchN3 = batchN.reshape(NP2 // _BN, 1, _BN)

    gcum = jnp.searchsorted(batch, jnp.arange(G + 1, dtype=batch.dtype)).astype(f32)
    npg = gcum[1:] - gcum[:-1]
    cnt_ln = jnp.maximum((npg + 1.0) * D, 1.0)
    cntn = jnp.maximum(npg, 1.0)

    x_aug = jnp.concatenate([x, cond, jnp.zeros((NTP - Nt, D), f32)], axis=0)

    esrc2 = esrc_p.reshape(_ESP // 128, 128)
    esrc2b = esrc_p.reshape(_ESP // 64, 64)
    edst2 = edst_gidx.reshape(_ESP // 128, 128)
    perm2 = jnp.clip(perm_p, 0, E - 1).reshape(_ESP // 128, 128)
    perm3 = perm_p.reshape(_ESP // _BE, 1, _BE)

    eattr_s = _fixattr(_sc_gather(edge_attr, perm2), perm3)
    la_sum = _sc_segsum(eattr_s.reshape(-1), edst_sort, bnd64,
                        64, 64, RB64, 512).reshape(NTP, 64)
    zeros_amx = jnp.zeros((NTP * AW,), f32)

    for p in params:
        st = _lnstats(x_aug, batch3)
        xn = _lnapply(x_aug, batch3, _stats8(st, cnt_ln))
        xs = _sc_gather(xn, esrc2)
        msgv = _msg(xs, eattr_s)
        aggs = _sc_segsum(msgv.reshape(-1), edst_sort, bnd64,
                          64, 64, RB64, 512).reshape(NTP, 64)
        q, kv, sk, epself, aself8 = _dense1(xn, aggs, la_sum, rcnt8, p)
        ep = _eproj(eattr_s, p['We'])
        kvs = _sc_gather(kv, esrc2b)
        qd = _sc_gather(q, edst2)
        alpha8 = _alpha(kvs, qd, ep)
        mx8 = _sc_segstat(alpha8.reshape(-1), edst_sort, bnd64, zeros_amx,
                          64, RB64, 1024, 'max').reshape(NTP, AW)
        amax8 = _mergeA(mx8, aself8)
        dens8 = _sc_segstat(alpha8.reshape(-1), edst_sort, bnd64,
                            amax8.reshape(-1), 64, RB64, 1024,
                            'expsum').reshape(NTP, AW)
        sm8, anself8 = _mergeB(dens8, aself8, amax8)
        smE = _sc_gather(sm8, edst2)
        tm = _tmsg(kvs, ep, alpha8, smE)
        touts = _sc_segsum(tm.reshape(-1), edst_sort, bnd128,
                           128, 128, RB128, 256).reshape(NTP, 128)
        l = _tout(touts, kv, epself, sk, anself8, p['Wlin'], p['blin'])
        st2 = _lnstats(l, batch3)
        x_aug = _mlpres(l, batch3, _stats8(st2, cnt_ln), x_aug, p)

    n_emb = x_aug[:N]
    pool = _pool(x_aug[:NP2], batchN3)
    glob = pool / cntn[:, None] + x_aug[N:Nt]
    NE = non_edge_index.shape[1]
    nep = _NEP - NE
    rows2 = jnp.concatenate([non_edge_index[0].astype(idt),
                             jnp.zeros((nep,), idt)]).reshape(_NEP // 128, 128)
    cols2 = jnp.concatenate([non_edge_index[1].astype(idt),
                             jnp.zeros((nep,), idt)]).reshape(_NEP // 128, 128)
    ne_emb = _sc_gather2_add(n_emb, rows2, cols2)[:NE]
    return n_emb, glob, ne_emb
